# baseline (speedup 1.0000x reference)
import jax
import jax.numpy as jnp
from jax.experimental import pallas as pl

_DEPTH = 1000
_GROUP = 16
_LANES = _GROUP * _DEPTH


def _onehot_body(idx_ref, out_ref):
    idx = idx_ref[...]
    col = jax.lax.broadcasted_iota(jnp.int32, (idx.shape[0], _DEPTH), 1)
    for s in range(_GROUP):
        seg = (idx[:, s:s + 1] == col).astype(jnp.float32)
        out_ref[:, s * _DEPTH:(s + 1) * _DEPTH] = seg


def kernel(indices):
    idx32 = indices.astype(jnp.int32)
    n, s = idx32.shape
    rows = n * s
    g = rows // _GROUP
    idx_g = idx32.reshape(g, _GROUP)
    blk = 128
    out = pl.pallas_call(
        _onehot_body,
        grid=(g // blk,),
        in_specs=[pl.BlockSpec((blk, _GROUP), lambda i: (i, 0))],
        out_specs=pl.BlockSpec((blk, _LANES), lambda i: (i, 0)),
        out_shape=jax.ShapeDtypeStruct((g, _LANES), jnp.float32),
    )(idx_g)
    return out.reshape(n, s, _DEPTH)


# speedup vs baseline: 1.5578x; 1.5578x over previous
import jax
import jax.numpy as jnp
from jax.experimental import pallas as pl
from jax.experimental.pallas import tpu as pltpu

_DEPTH = 1000
_BLK = 64
_NBUF = 6


def _onehot_body(idx_ref, out_ref, scratch_ref, sem_ref):
    i = pl.program_id(0)
    nblocks = pl.num_programs(0)
    slot = jax.lax.rem(i, _NBUF)

    @pl.when(i >= _NBUF)
    def _wait_prev():
        pltpu.make_async_copy(
            scratch_ref.at[slot], out_ref.at[pl.ds(0, _BLK)], sem_ref.at[slot]
        ).wait()

    idx = idx_ref[...]
    b, s = idx.shape
    iota = jax.lax.broadcasted_iota(jnp.int32, (b, s, _DEPTH), 2)
    scratch_ref[slot] = (iota == idx[:, :, None]).astype(jnp.float32)

    pltpu.make_async_copy(
        scratch_ref.at[slot], out_ref.at[pl.ds(i * _BLK, _BLK)], sem_ref.at[slot]
    ).start()

    @pl.when(i == nblocks - 1)
    def _drain():
        for k in range(_NBUF):
            pltpu.make_async_copy(
                scratch_ref.at[k], out_ref.at[pl.ds(0, _BLK)], sem_ref.at[k]
            ).wait()


def kernel(indices):
    idx32 = indices.astype(jnp.int32)
    n, s = idx32.shape
    out = pl.pallas_call(
        _onehot_body,
        grid=(n // _BLK,),
        in_specs=[pl.BlockSpec((_BLK, s), lambda i: (i, 0))],
        out_specs=pl.BlockSpec(memory_space=pl.ANY),
        out_shape=jax.ShapeDtypeStruct((n, s, _DEPTH), jnp.float32),
        scratch_shapes=[
            pltpu.VMEM((_NBUF, _BLK, s, _DEPTH), jnp.float32),
            pltpu.SemaphoreType.DMA((_NBUF,)),
        ],
    )(idx32)
    return out
